# baseline (speedup 1.0000x reference)
import functools

import jax
import jax.numpy as jnp
from jax import lax
from jax.experimental import pallas as pl
from jax.experimental.pallas import tpu as pltpu
from jax.experimental.pallas import tpu_sc as plsc

B = 16
R = 64
C = 256
N = 32768
HW = R * R
CHW = C // 2
SUB = 128
NSUB = N // SUB
NTILE = 16
NCORE = 2
KPT = NSUB // NTILE
BPC = B // NCORE
ROWS_PT = HW // NTILE


def _sc_body(feats_hbm, zeros_hbm, bidx_hbm, sidx_hbm, out_hbm,
             fbuf, zbuf, bbuf, sbuf, ibuf, acc):
    cid = lax.axis_index("c")
    tid = lax.axis_index("s")

    pltpu.sync_copy(zeros_hbm, zbuf)
    for k in range(KPT):
        chunk = k * NTILE + tid
        pltpu.sync_copy(bidx_hbm.at[pl.ds(chunk * SUB, SUB)],
                        bbuf.at[pl.ds(k * SUB, SUB)])
        pltpu.sync_copy(sidx_hbm.at[pl.ds(chunk * SUB, SUB)],
                        sbuf.at[pl.ds(k * SUB, SUB)])

    def batch_body(bl, carry):
        b = cid * BPC + bl
        r0 = tid * ROWS_PT
        for c0 in (0, CHW):
            pltpu.sync_copy(zbuf, acc.at[pl.ds(r0, SUB)])
            pltpu.sync_copy(zbuf, acc.at[pl.ds(r0 + SUB, SUB)])
            plsc.subcore_barrier()

            def sub_body(k, carry2):
                base = k * SUB
                bfirst = bbuf[pl.ds(base, 16)][0]
                blast = bbuf[pl.ds(base + SUB - 16, 16)][15]

                @pl.when(jnp.logical_and(bfirst <= b, b <= blast))
                def _():
                    chunk = k * NTILE + tid
                    pltpu.sync_copy(
                        feats_hbm.at[pl.ds(chunk * SUB, SUB), pl.ds(c0, CHW)],
                        fbuf)
                    for j in range(SUB // 16):
                        vb = bbuf[pl.ds(base + j * 16, 16)]
                        vs = sbuf[pl.ds(base + j * 16, 16)]
                        ibuf[pl.ds(j * 16, 16)] = jnp.where(
                            vb == b, vs, HW + tid)
                    pltpu.sync_copy(fbuf, acc.at[ibuf], add=True)

                return carry2

            lax.fori_loop(0, KPT, sub_body, 0)
            plsc.subcore_barrier()
            pltpu.sync_copy(acc.at[pl.ds(r0, ROWS_PT)],
                            out_hbm.at[b, pl.ds(r0, ROWS_PT), pl.ds(c0, CHW)])
        return carry

    lax.fori_loop(0, BPC, batch_body, 0)


_scatter_sc = functools.partial(
    pl.kernel,
    out_type=jax.ShapeDtypeStruct((B, HW, C), jnp.float32),
    mesh=plsc.VectorSubcoreMesh(core_axis_name="c", subcore_axis_name="s"),
    scratch_types=[
        pltpu.VMEM((SUB, CHW), jnp.float32),
        pltpu.VMEM((SUB, CHW), jnp.float32),
        pltpu.VMEM((KPT * SUB,), jnp.int32),
        pltpu.VMEM((KPT * SUB,), jnp.int32),
        pltpu.VMEM((SUB,), jnp.int32),
        pltpu.VMEM_SHARED((HW + NTILE, CHW), jnp.float32),
    ],
)(_sc_body)


TH = 512


def _t_body(in_ref, out_ref):
    out_ref[0] = in_ref[0].T


_transpose_tc = pl.pallas_call(
    _t_body,
    grid=(B, HW // TH),
    in_specs=[pl.BlockSpec((1, TH, C), lambda b, j: (b, j, 0))],
    out_specs=pl.BlockSpec((1, C, TH), lambda b, j: (b, 0, j)),
    out_shape=jax.ShapeDtypeStruct((B, C, HW), jnp.float32),
)


@jax.jit
def kernel(feats, batch_idx, spatial_idx):
    bidx = batch_idx.astype(jnp.int32)
    sidx = spatial_idx.astype(jnp.int32)
    zeros = jnp.zeros((SUB, CHW), jnp.float32)
    dense = _scatter_sc(feats, zeros, bidx, sidx)
    out = _transpose_tc(dense)
    return out.reshape(B, C, R, R)
